# baseline (speedup 1.0000x reference)
import functools

import jax
import jax.numpy as jnp
from jax import lax
from jax.experimental import pallas as pl
from jax.experimental.pallas import tpu as pltpu
from jax.experimental.pallas import tpu_sc as plsc

N_NODES = 10000
E = 320000
NC, NS = 2, 16
NW = NC * NS
EPT = 10240
EPAD = NW * EPT
NROWS = 10008
NZT = 8
ZR = NROWS // NZT
BR = 1000
DW = 8

_mesh = plsc.VectorSubcoreMesh(
    core_axis_name="c", subcore_axis_name="s", num_cores=NC, num_subcores=NS)

_params = pltpu.CompilerParams(use_tc_tiling_on_sc=False)


DEG_CHUNK = 64
DEG_CPT = EPT // DEG_CHUNK


@functools.partial(
    pl.kernel,
    out_type=jax.ShapeDtypeStruct((NC * NROWS, DW), jnp.float32),
    mesh=_mesh,
    compiler_params=_params,
    scratch_types=[
        pltpu.VMEM((2, DEG_CHUNK), jnp.int32),
        pltpu.VMEM((2, DEG_CHUNK), jnp.int32),
        pltpu.VMEM((DEG_CHUNK, DW), jnp.float32),
        pltpu.VMEM_SHARED((NROWS, DW), jnp.float32),
        pltpu.SemaphoreType.DMA,
        pltpu.SemaphoreType.DMA,
    ],
)
def _deg_kernel(sd_hbm, ones_hbm, zero_hbm, out_hbm,
                ib0, ib1, ones_v, deg_sh, si0, si1):
    cid = lax.axis_index("c")
    tid = lax.axis_index("s")
    wid = cid * NS + tid
    pltpu.sync_copy(ones_hbm, ones_v)

    @pl.when(tid < NZT)
    def _zero():
        pltpu.sync_copy(zero_hbm, deg_sh.at[pl.ds(tid * ZR, ZR)])

    plsc.subcore_barrier()

    pltpu.async_copy(sd_hbm.at[wid, 0], ib0, si0)
    pltpu.async_copy(sd_hbm.at[wid, 1], ib1, si1)

    def body(j2, carry):
        a = 2 * j2
        pltpu.make_async_copy(sd_hbm.at[wid, a], ib0, si0).wait()
        pltpu.sync_copy(ones_v, deg_sh.at[ib0.at[1]], add=True)
        pltpu.async_copy(
            sd_hbm.at[wid, jnp.minimum(a + 2, DEG_CPT - 1)], ib0, si0)
        pltpu.make_async_copy(sd_hbm.at[wid, a + 1], ib1, si1).wait()
        pltpu.sync_copy(ones_v, deg_sh.at[ib1.at[1]], add=True)
        pltpu.async_copy(
            sd_hbm.at[wid, jnp.minimum(a + 3, DEG_CPT - 1)], ib1, si1)
        return carry

    lax.fori_loop(0, DEG_CPT // 2, body, 0)
    pltpu.make_async_copy(sd_hbm.at[wid, DEG_CPT - 1], ib0, si0).wait()
    pltpu.make_async_copy(sd_hbm.at[wid, DEG_CPT - 1], ib1, si1).wait()

    plsc.subcore_barrier()

    @pl.when(tid < NZT)
    def _dump():
        pltpu.sync_copy(
            deg_sh.at[pl.ds(tid * ZR, ZR)],
            out_hbm.at[pl.ds(cid * NROWS + tid * ZR, ZR)],
        )


def _make_agg(D, CHUNK):
    CPT = EPT // CHUNK

    @functools.partial(
        pl.kernel,
        out_type=jax.ShapeDtypeStruct((NC * NROWS, D), jnp.float32),
        mesh=_mesh,
        compiler_params=_params,
        scratch_types=[
            pltpu.VMEM((2, CHUNK), jnp.int32),
            pltpu.VMEM((2, CHUNK), jnp.int32),
            pltpu.VMEM((CHUNK, D), jnp.float32),
            pltpu.VMEM((CHUNK, D), jnp.float32),
            pltpu.VMEM_SHARED((NROWS, D), jnp.float32),
            pltpu.SemaphoreType.DMA,
            pltpu.SemaphoreType.DMA,
            pltpu.SemaphoreType.DMA,
            pltpu.SemaphoreType.DMA,
        ],
    )
    def _agg_kernel(tbl_hbm, sd_hbm, zero_hbm, out_hbm,
                    ib0, ib1, buf0, buf1, acc_sh, si0, si1, sg0, sg1):
        cid = lax.axis_index("c")
        tid = lax.axis_index("s")
        wid = cid * NS + tid

        @pl.when(tid < NZT)
        def _zero():
            pltpu.sync_copy(zero_hbm, acc_sh.at[pl.ds(tid * ZR, ZR)])

        plsc.subcore_barrier()

        pltpu.sync_copy(sd_hbm.at[wid, 0], ib0)
        pltpu.async_copy(tbl_hbm.at[ib0.at[0]], buf0, sg0)
        pltpu.async_copy(sd_hbm.at[wid, 1], ib1, si1)

        def body(j2, carry):
            a = 2 * j2
            nxt0 = jnp.minimum(a + 2, CPT - 1)
            nxt1 = jnp.minimum(a + 3, CPT - 1)
            pltpu.make_async_copy(sd_hbm.at[wid, a + 1], ib1, si1).wait()
            pltpu.make_async_copy(tbl_hbm.at[ib0.at[0]], buf0, sg0).wait()
            pltpu.async_copy(tbl_hbm.at[ib1.at[0]], buf1, sg1)
            pltpu.sync_copy(buf0, acc_sh.at[ib0.at[1]], add=True)
            pltpu.async_copy(sd_hbm.at[wid, nxt0], ib0, si0)

            pltpu.make_async_copy(sd_hbm.at[wid, nxt0], ib0, si0).wait()
            pltpu.make_async_copy(tbl_hbm.at[ib1.at[0]], buf1, sg1).wait()
            pltpu.async_copy(tbl_hbm.at[ib0.at[0]], buf0, sg0)
            pltpu.sync_copy(buf1, acc_sh.at[ib1.at[1]], add=True)
            pltpu.async_copy(sd_hbm.at[wid, nxt1], ib1, si1)
            return carry

        lax.fori_loop(0, CPT // 2, body, 0)
        pltpu.make_async_copy(tbl_hbm.at[ib0.at[0]], buf0, sg0).wait()
        pltpu.make_async_copy(sd_hbm.at[wid, CPT - 1], ib1, si1).wait()

        plsc.subcore_barrier()

        @pl.when(tid < NZT)
        def _dump():
            pltpu.sync_copy(
                acc_sh.at[pl.ds(tid * ZR, ZR)],
                out_hbm.at[pl.ds(cid * NROWS + tid * ZR, ZR)],
            )

    return _agg_kernel


def _make_agg_ring(D, CHUNK, CPT0, CPT1):
    assert CPT0 % 8 == 0 and CPT1 % 8 == 0

    @functools.partial(
        pl.kernel,
        out_type=jax.ShapeDtypeStruct((NC * NROWS, D), jnp.float32),
        mesh=_mesh,
        compiler_params=_params,
        scratch_types=(
            [pltpu.VMEM((2, CHUNK), jnp.int32)] * 8
            + [pltpu.VMEM((CHUNK, D), jnp.float32)] * 4
            + [pltpu.VMEM_SHARED((NROWS, D), jnp.float32)]
            + [pltpu.SemaphoreType.DMA] * 12
        ),
    )
    def _agg_kernel(tbl_hbm, sd_hbm, zero_hbm, out_hbm, *sc):
        jb = sc[0:8]
        buf = sc[8:12]
        acc_sh = sc[12]
        si = sc[13:21]
        sg = sc[21:25]
        cid = lax.axis_index("c")
        tid = lax.axis_index("s")
        base = jnp.where(cid == 0, tid * CPT0, NS * CPT0 + tid * CPT1)
        cpt = jnp.where(cid == 0, CPT0, CPT1)
        last = base + cpt - 1

        @pl.when(tid < NZT)
        def _zero():
            pltpu.sync_copy(zero_hbm, acc_sh.at[pl.ds(tid * ZR, ZR)])

        plsc.subcore_barrier()

        for t in range(8):
            pltpu.async_copy(sd_hbm.at[base + t], jb[t], si[t])
        for r in range(4):
            pltpu.make_async_copy(sd_hbm.at[base + r], jb[r], si[r]).wait()
            pltpu.async_copy(tbl_hbm.at[jb[r].at[0]], buf[r], sg[r])

        def body(k, carry):
            c0 = base + 8 * k
            for r in range(8):
                c = c0 + r
                pltpu.make_async_copy(
                    tbl_hbm.at[jb[r].at[0]], buf[r % 4], sg[r % 4]).wait()
                pltpu.sync_copy(buf[r % 4], acc_sh.at[jb[r].at[1]], add=True)
                pltpu.make_async_copy(
                    sd_hbm.at[c], jb[(r + 4) % 8], si[(r + 4) % 8]).wait()
                pltpu.async_copy(
                    tbl_hbm.at[jb[(r + 4) % 8].at[0]], buf[r % 4], sg[r % 4])
                pltpu.async_copy(
                    sd_hbm.at[jnp.minimum(c + 8, last)], jb[r], si[r])
            return carry

        lax.fori_loop(0, cpt // 8, body, 0)
        for r in range(4):
            pltpu.make_async_copy(
                tbl_hbm.at[jb[r].at[0]], buf[r], sg[r]).wait()
        for r in range(4, 8):
            pltpu.make_async_copy(sd_hbm.at[last], jb[r], si[r]).wait()

        plsc.subcore_barrier()

        @pl.when(tid < NZT)
        def _dump():
            pltpu.sync_copy(
                acc_sh.at[pl.ds(tid * ZR, ZR)],
                out_hbm.at[pl.ds(cid * NROWS + tid * ZR, ZR)],
            )

    return _agg_kernel


AGG_CHUNK = 32
NCH = EPAD // AGG_CHUNK // NS
CPT128_0, CPT128_1 = 144, 496
CPT32_0, CPT32_1 = 200, 440
assert CPT128_0 + CPT128_1 == NCH and CPT32_0 + CPT32_1 == NCH
_agg128 = _make_agg_ring(128, AGG_CHUNK, CPT128_0, CPT128_1)
_agg32 = _make_agg_ring(32, AGG_CHUNK, CPT32_0, CPT32_1)


def _stage_a(degp, x):
    def body(degp_ref, x_ref, y_ref, dinv_ref):
        deg = degp_ref[0] + degp_ref[1] + 1.0
        dinv = lax.rsqrt(deg)
        dinv_ref[...] = dinv
        y_ref[...] = x_ref[...] * dinv[:, :1]

    return pl.pallas_call(
        body,
        grid=(N_NODES // BR,),
        in_specs=[
            pl.BlockSpec((2, BR, DW), lambda i: (0, i, 0)),
            pl.BlockSpec((BR, 128), lambda i: (i, 0)),
        ],
        out_specs=[
            pl.BlockSpec((BR, 128), lambda i: (i, 0)),
            pl.BlockSpec((BR, DW), lambda i: (i, 0)),
        ],
        out_shape=[
            jax.ShapeDtypeStruct((N_NODES, 128), jnp.float32),
            jax.ShapeDtypeStruct((N_NODES, DW), jnp.float32),
        ],
    )(degp, x)


def _stage_b(p, y, dinv8, W1, b1, W2):
    def body(p_ref, y_ref, dinv_ref, W1_ref, b1_ref, W2_ref, u_ref):
        dinv = dinv_ref[:, :1]
        agg = (p_ref[0] + p_ref[1] + y_ref[...]) * dinv
        h = jnp.dot(agg, W1_ref[...], preferred_element_type=jnp.float32)
        h = jnp.maximum(h + b1_ref[...], 0.0)
        t = jnp.dot(h, W2_ref[...], preferred_element_type=jnp.float32)
        u_ref[...] = t * dinv

    return pl.pallas_call(
        body,
        grid=(N_NODES // BR,),
        in_specs=[
            pl.BlockSpec((2, BR, 128), lambda i: (0, i, 0)),
            pl.BlockSpec((BR, 128), lambda i: (i, 0)),
            pl.BlockSpec((BR, DW), lambda i: (i, 0)),
            pl.BlockSpec((128, 256), lambda i: (0, 0)),
            pl.BlockSpec((1, 256), lambda i: (0, 0)),
            pl.BlockSpec((256, 32), lambda i: (0, 0)),
        ],
        out_specs=pl.BlockSpec((BR, 32), lambda i: (i, 0)),
        out_shape=jax.ShapeDtypeStruct((N_NODES, 32), jnp.float32),
    )(p, y, dinv8, W1, b1, W2)


def _stage_c(q, u, dinv8, b2):
    def body(q_ref, u_ref, dinv_ref, b2_ref, z_ref):
        dinv = dinv_ref[:, :1]
        z_ref[...] = (q_ref[0] + q_ref[1] + u_ref[...]) * dinv + b2_ref[...]

    return pl.pallas_call(
        body,
        grid=(N_NODES // BR,),
        in_specs=[
            pl.BlockSpec((2, BR, 32), lambda i: (0, i, 0)),
            pl.BlockSpec((BR, 32), lambda i: (i, 0)),
            pl.BlockSpec((BR, DW), lambda i: (i, 0)),
            pl.BlockSpec((1, 32), lambda i: (0, 0)),
        ],
        out_specs=pl.BlockSpec((BR, 32), lambda i: (i, 0)),
        out_shape=jax.ShapeDtypeStruct((N_NODES, 32), jnp.float32),
    )(q, u, dinv8, b2)


def _make_sd(spad, dpad, chunk):
    cpt = EPT // chunk
    return jnp.stack(
        [spad.reshape(NW, cpt, chunk), dpad.reshape(NW, cpt, chunk)], axis=2)


def kernel(x, edge_index, W1, b1, W2, b2):
    src = edge_index[0].astype(jnp.int32)
    dst = edge_index[1].astype(jnp.int32)
    pad = EPAD - E
    spad = jnp.concatenate([src, jnp.zeros((pad,), jnp.int32)])
    dpad = jnp.concatenate([dst, jnp.full((pad,), N_NODES, jnp.int32)])
    sd64 = _make_sd(spad, dpad, DEG_CHUNK)
    sd32 = jnp.stack(
        [spad.reshape(-1, AGG_CHUNK), dpad.reshape(-1, AGG_CHUNK)], axis=1)

    ones = jnp.ones((DEG_CHUNK, DW), jnp.float32)
    zd = jnp.zeros((ZR, DW), jnp.float32)
    z128 = jnp.zeros((ZR, 128), jnp.float32)
    z32 = jnp.zeros((ZR, 32), jnp.float32)

    degp = _deg_kernel(sd64, ones, zd).reshape(NC, NROWS, DW)[:, :N_NODES]
    y, dinv8 = _stage_a(degp, x)
    p = _agg128(y, sd32, z128).reshape(NC, NROWS, 128)[:, :N_NODES]
    u = _stage_b(p, y, dinv8, W1, b1.reshape(1, -1), W2)
    q = _agg32(u, sd32, z32).reshape(NC, NROWS, 32)[:, :N_NODES]
    z = _stage_c(q, u, dinv8, b2.reshape(1, -1))
    return z


# speedup vs baseline: 1.1035x; 1.1035x over previous
import functools

import jax
import jax.numpy as jnp
from jax import lax
from jax.experimental import pallas as pl
from jax.experimental.pallas import tpu as pltpu
from jax.experimental.pallas import tpu_sc as plsc

N_NODES = 10000
E = 320000
NC, NS = 2, 16
NW = NC * NS
EPT = 10240
EPAD = NW * EPT
NROWS = 10008
NZT = 8
ZR = NROWS // NZT
BR = 1000
DW = 8

_mesh = plsc.VectorSubcoreMesh(
    core_axis_name="c", subcore_axis_name="s", num_cores=NC, num_subcores=NS)

_params = pltpu.CompilerParams(use_tc_tiling_on_sc=False)


DEG_CHUNK = 64
DEG_CPT = EPT // DEG_CHUNK


@functools.partial(
    pl.kernel,
    out_type=jax.ShapeDtypeStruct((NC * NROWS, DW), jnp.float32),
    mesh=_mesh,
    compiler_params=_params,
    scratch_types=[
        pltpu.VMEM((2, DEG_CHUNK), jnp.int32),
        pltpu.VMEM((2, DEG_CHUNK), jnp.int32),
        pltpu.VMEM((DEG_CHUNK, DW), jnp.float32),
        pltpu.VMEM_SHARED((NROWS, DW), jnp.float32),
        pltpu.SemaphoreType.DMA,
        pltpu.SemaphoreType.DMA,
    ],
)
def _deg_kernel(sd_hbm, ones_hbm, zero_hbm, out_hbm,
                ib0, ib1, ones_v, deg_sh, si0, si1):
    cid = lax.axis_index("c")
    tid = lax.axis_index("s")
    wid = cid * NS + tid
    pltpu.sync_copy(ones_hbm, ones_v)

    @pl.when(tid < NZT)
    def _zero():
        pltpu.sync_copy(zero_hbm, deg_sh.at[pl.ds(tid * ZR, ZR)])

    plsc.subcore_barrier()

    pltpu.async_copy(sd_hbm.at[wid, 0], ib0, si0)
    pltpu.async_copy(sd_hbm.at[wid, 1], ib1, si1)

    def body(j2, carry):
        a = 2 * j2
        pltpu.make_async_copy(sd_hbm.at[wid, a], ib0, si0).wait()
        pltpu.sync_copy(ones_v, deg_sh.at[ib0.at[1]], add=True)
        pltpu.async_copy(
            sd_hbm.at[wid, jnp.minimum(a + 2, DEG_CPT - 1)], ib0, si0)
        pltpu.make_async_copy(sd_hbm.at[wid, a + 1], ib1, si1).wait()
        pltpu.sync_copy(ones_v, deg_sh.at[ib1.at[1]], add=True)
        pltpu.async_copy(
            sd_hbm.at[wid, jnp.minimum(a + 3, DEG_CPT - 1)], ib1, si1)
        return carry

    lax.fori_loop(0, DEG_CPT // 2, body, 0)
    pltpu.make_async_copy(sd_hbm.at[wid, DEG_CPT - 1], ib0, si0).wait()
    pltpu.make_async_copy(sd_hbm.at[wid, DEG_CPT - 1], ib1, si1).wait()

    plsc.subcore_barrier()

    @pl.when(tid < NZT)
    def _dump():
        pltpu.sync_copy(
            deg_sh.at[pl.ds(tid * ZR, ZR)],
            out_hbm.at[pl.ds(cid * NROWS + tid * ZR, ZR)],
        )


def _make_agg(D, CHUNK):
    CPT = EPT // CHUNK

    @functools.partial(
        pl.kernel,
        out_type=jax.ShapeDtypeStruct((NC * NROWS, D), jnp.float32),
        mesh=_mesh,
        compiler_params=_params,
        scratch_types=[
            pltpu.VMEM((2, CHUNK), jnp.int32),
            pltpu.VMEM((2, CHUNK), jnp.int32),
            pltpu.VMEM((CHUNK, D), jnp.float32),
            pltpu.VMEM((CHUNK, D), jnp.float32),
            pltpu.VMEM_SHARED((NROWS, D), jnp.float32),
            pltpu.SemaphoreType.DMA,
            pltpu.SemaphoreType.DMA,
            pltpu.SemaphoreType.DMA,
            pltpu.SemaphoreType.DMA,
        ],
    )
    def _agg_kernel(tbl_hbm, sd_hbm, zero_hbm, out_hbm,
                    ib0, ib1, buf0, buf1, acc_sh, si0, si1, sg0, sg1):
        cid = lax.axis_index("c")
        tid = lax.axis_index("s")
        wid = cid * NS + tid

        @pl.when(tid < NZT)
        def _zero():
            pltpu.sync_copy(zero_hbm, acc_sh.at[pl.ds(tid * ZR, ZR)])

        plsc.subcore_barrier()

        pltpu.sync_copy(sd_hbm.at[wid, 0], ib0)
        pltpu.async_copy(tbl_hbm.at[ib0.at[0]], buf0, sg0)
        pltpu.async_copy(sd_hbm.at[wid, 1], ib1, si1)

        def body(j2, carry):
            a = 2 * j2
            nxt0 = jnp.minimum(a + 2, CPT - 1)
            nxt1 = jnp.minimum(a + 3, CPT - 1)
            pltpu.make_async_copy(sd_hbm.at[wid, a + 1], ib1, si1).wait()
            pltpu.make_async_copy(tbl_hbm.at[ib0.at[0]], buf0, sg0).wait()
            pltpu.async_copy(tbl_hbm.at[ib1.at[0]], buf1, sg1)
            pltpu.sync_copy(buf0, acc_sh.at[ib0.at[1]], add=True)
            pltpu.async_copy(sd_hbm.at[wid, nxt0], ib0, si0)

            pltpu.make_async_copy(sd_hbm.at[wid, nxt0], ib0, si0).wait()
            pltpu.make_async_copy(tbl_hbm.at[ib1.at[0]], buf1, sg1).wait()
            pltpu.async_copy(tbl_hbm.at[ib0.at[0]], buf0, sg0)
            pltpu.sync_copy(buf1, acc_sh.at[ib1.at[1]], add=True)
            pltpu.async_copy(sd_hbm.at[wid, nxt1], ib1, si1)
            return carry

        lax.fori_loop(0, CPT // 2, body, 0)
        pltpu.make_async_copy(tbl_hbm.at[ib0.at[0]], buf0, sg0).wait()
        pltpu.make_async_copy(sd_hbm.at[wid, CPT - 1], ib1, si1).wait()

        plsc.subcore_barrier()

        @pl.when(tid < NZT)
        def _dump():
            pltpu.sync_copy(
                acc_sh.at[pl.ds(tid * ZR, ZR)],
                out_hbm.at[pl.ds(cid * NROWS + tid * ZR, ZR)],
            )

    return _agg_kernel


def _make_agg_ring(D, CHUNK, CPT0, CPT1):
    assert CPT0 % 8 == 0 and CPT1 % 8 == 0

    @functools.partial(
        pl.kernel,
        out_type=jax.ShapeDtypeStruct((NC * NROWS, D), jnp.float32),
        mesh=_mesh,
        compiler_params=_params,
        scratch_types=(
            [pltpu.VMEM((2, CHUNK), jnp.int32)] * 8
            + [pltpu.VMEM((CHUNK, D), jnp.float32)] * 4
            + [pltpu.VMEM_SHARED((NROWS, D), jnp.float32)]
            + [pltpu.SemaphoreType.DMA] * 12
        ),
    )
    def _agg_kernel(tbl_hbm, sd_hbm, zero_hbm, out_hbm, *sc):
        jb = sc[0:8]
        buf = sc[8:12]
        acc_sh = sc[12]
        si = sc[13:21]
        sg = sc[21:25]
        cid = lax.axis_index("c")
        tid = lax.axis_index("s")
        base = jnp.where(cid == 0, tid * CPT0, NS * CPT0 + tid * CPT1)
        cpt = jnp.where(cid == 0, CPT0, CPT1)
        last = base + cpt - 1

        @pl.when(tid < NZT)
        def _zero():
            pltpu.sync_copy(zero_hbm, acc_sh.at[pl.ds(tid * ZR, ZR)])

        plsc.subcore_barrier()

        for t in range(8):
            pltpu.async_copy(sd_hbm.at[base + t], jb[t], si[t])
        for r in range(4):
            pltpu.make_async_copy(sd_hbm.at[base + r], jb[r], si[r]).wait()
            pltpu.async_copy(tbl_hbm.at[jb[r].at[0]], buf[r], sg[r])

        def body(k, carry):
            c0 = base + 8 * k
            for r in range(8):
                c = c0 + r
                pltpu.make_async_copy(
                    tbl_hbm.at[jb[r].at[0]], buf[r % 4], sg[r % 4]).wait()
                pltpu.sync_copy(buf[r % 4], acc_sh.at[jb[r].at[1]], add=True)
                pltpu.make_async_copy(
                    sd_hbm.at[c], jb[(r + 4) % 8], si[(r + 4) % 8]).wait()
                pltpu.async_copy(
                    tbl_hbm.at[jb[(r + 4) % 8].at[0]], buf[r % 4], sg[r % 4])
                pltpu.async_copy(
                    sd_hbm.at[jnp.minimum(c + 8, last)], jb[r], si[r])
            return carry

        lax.fori_loop(0, cpt // 8, body, 0)
        for r in range(4):
            pltpu.make_async_copy(
                tbl_hbm.at[jb[r].at[0]], buf[r], sg[r]).wait()
        for r in range(4, 8):
            pltpu.make_async_copy(sd_hbm.at[last], jb[r], si[r]).wait()

        plsc.subcore_barrier()

        @pl.when(tid < NZT)
        def _dump():
            pltpu.sync_copy(
                acc_sh.at[pl.ds(tid * ZR, ZR)],
                out_hbm.at[pl.ds(cid * NROWS + tid * ZR, ZR)],
            )

    return _agg_kernel


AGG_CHUNK = 32
NCH = EPAD // AGG_CHUNK // NS
CPT128_0, CPT128_1 = 496, 144
CPT32_0, CPT32_1 = 440, 200
assert CPT128_0 + CPT128_1 == NCH and CPT32_0 + CPT32_1 == NCH
_agg128 = _make_agg_ring(128, AGG_CHUNK, CPT128_0, CPT128_1)
_agg32 = _make_agg_ring(32, AGG_CHUNK, CPT32_0, CPT32_1)


def _stage_a(degp, x):
    def body(degp_ref, x_ref, y_ref, dinv_ref):
        deg = degp_ref[0] + degp_ref[1] + 1.0
        dinv = lax.rsqrt(deg)
        dinv_ref[...] = dinv
        y_ref[...] = x_ref[...] * dinv[:, :1]

    return pl.pallas_call(
        body,
        grid=(N_NODES // BR,),
        in_specs=[
            pl.BlockSpec((2, BR, DW), lambda i: (0, i, 0)),
            pl.BlockSpec((BR, 128), lambda i: (i, 0)),
        ],
        out_specs=[
            pl.BlockSpec((BR, 128), lambda i: (i, 0)),
            pl.BlockSpec((BR, DW), lambda i: (i, 0)),
        ],
        out_shape=[
            jax.ShapeDtypeStruct((N_NODES, 128), jnp.float32),
            jax.ShapeDtypeStruct((N_NODES, DW), jnp.float32),
        ],
    )(degp, x)


def _stage_b(p, y, dinv8, W1, b1, W2):
    def body(p_ref, y_ref, dinv_ref, W1_ref, b1_ref, W2_ref, u_ref):
        dinv = dinv_ref[:, :1]
        agg = (p_ref[0] + p_ref[1] + y_ref[...]) * dinv
        h = jnp.dot(agg, W1_ref[...], preferred_element_type=jnp.float32)
        h = jnp.maximum(h + b1_ref[...], 0.0)
        t = jnp.dot(h, W2_ref[...], preferred_element_type=jnp.float32)
        u_ref[...] = t * dinv

    return pl.pallas_call(
        body,
        grid=(N_NODES // BR,),
        in_specs=[
            pl.BlockSpec((2, BR, 128), lambda i: (0, i, 0)),
            pl.BlockSpec((BR, 128), lambda i: (i, 0)),
            pl.BlockSpec((BR, DW), lambda i: (i, 0)),
            pl.BlockSpec((128, 256), lambda i: (0, 0)),
            pl.BlockSpec((1, 256), lambda i: (0, 0)),
            pl.BlockSpec((256, 32), lambda i: (0, 0)),
        ],
        out_specs=pl.BlockSpec((BR, 32), lambda i: (i, 0)),
        out_shape=jax.ShapeDtypeStruct((N_NODES, 32), jnp.float32),
    )(p, y, dinv8, W1, b1, W2)


def _stage_c(q, u, dinv8, b2):
    def body(q_ref, u_ref, dinv_ref, b2_ref, z_ref):
        dinv = dinv_ref[:, :1]
        z_ref[...] = (q_ref[0] + q_ref[1] + u_ref[...]) * dinv + b2_ref[...]

    return pl.pallas_call(
        body,
        grid=(N_NODES // BR,),
        in_specs=[
            pl.BlockSpec((2, BR, 32), lambda i: (0, i, 0)),
            pl.BlockSpec((BR, 32), lambda i: (i, 0)),
            pl.BlockSpec((BR, DW), lambda i: (i, 0)),
            pl.BlockSpec((1, 32), lambda i: (0, 0)),
        ],
        out_specs=pl.BlockSpec((BR, 32), lambda i: (i, 0)),
        out_shape=jax.ShapeDtypeStruct((N_NODES, 32), jnp.float32),
    )(q, u, dinv8, b2)


def _make_sd(spad, dpad, chunk):
    cpt = EPT // chunk
    return jnp.stack(
        [spad.reshape(NW, cpt, chunk), dpad.reshape(NW, cpt, chunk)], axis=2)


def kernel(x, edge_index, W1, b1, W2, b2):
    src = edge_index[0].astype(jnp.int32)
    dst = edge_index[1].astype(jnp.int32)
    pad = EPAD - E
    spad = jnp.concatenate([src, jnp.zeros((pad,), jnp.int32)])
    dpad = jnp.concatenate([dst, jnp.full((pad,), N_NODES, jnp.int32)])
    sd64 = _make_sd(spad, dpad, DEG_CHUNK)
    sd32 = jnp.stack(
        [spad.reshape(-1, AGG_CHUNK), dpad.reshape(-1, AGG_CHUNK)], axis=1)

    ones = jnp.ones((DEG_CHUNK, DW), jnp.float32)
    zd = jnp.zeros((ZR, DW), jnp.float32)
    z128 = jnp.zeros((ZR, 128), jnp.float32)
    z32 = jnp.zeros((ZR, 32), jnp.float32)

    degp = _deg_kernel(sd64, ones, zd).reshape(NC, NROWS, DW)[:, :N_NODES]
    y, dinv8 = _stage_a(degp, x)
    p = _agg128(y, sd32, z128).reshape(NC, NROWS, 128)[:, :N_NODES]
    u = _stage_b(p, y, dinv8, W1, b1.reshape(1, -1), W2)
    q = _agg32(u, sd32, z32).reshape(NC, NROWS, 32)[:, :N_NODES]
    z = _stage_c(q, u, dinv8, b2.reshape(1, -1))
    return z
